# Initial kernel scaffold; baseline (speedup 1.0000x reference)
import jax
import jax.numpy as jnp
from jax.experimental import pallas as pl


def kernel(x, t, z, edge_index, W_gc0, b_gc0, W_gc1, b_gc1, W_t00_0, b_t00_0, W_t00_1, b_t00_1, W_t10_0, b_t10_0, W_t10_1, b_t10_1, W_t01, b_t01, W_t11, b_t11, W_pp, b_pp):
    raise NotImplementedError("write your pallas kernel here")


# speedup vs baseline: 10.3415x; 10.3415x over previous
import functools

import jax
import jax.numpy as jnp
from jax import lax
from jax.experimental import pallas as pl
from jax.experimental.pallas import tpu as pltpu
from jax.experimental.pallas import tpu_sc as plsc

NC = 2
NS = 16
LANES = 16
CHUNK = 128


def _fill_vmem_zeros(ref, rows, cols):
    per_row = cols // LANES

    def body(i, carry):
        r = i // per_row
        j = i % per_row
        ref[r, pl.ds(j * LANES, LANES)] = jnp.zeros((LANES,), jnp.float32)
        return carry

    lax.fori_loop(0, rows * per_row, body, None)


@functools.lru_cache(maxsize=None)
def _make_sc_kernels(n_pad, e_pad, d):
    rows_per_sub = n_pad // NS
    edges_per_worker = e_pad // (NC * NS)
    n_chunks = edges_per_worker // CHUNK
    mesh = plsc.VectorSubcoreMesh(core_axis_name="c", subcore_axis_name="s")

    @functools.partial(
        pl.kernel,
        out_type=jax.ShapeDtypeStruct((NC, n_pad), jnp.float32),
        mesh=mesh,
        scratch_types=[
            pltpu.VMEM((CHUNK,), jnp.int32),
            pltpu.VMEM((CHUNK,), jnp.float32),
            pltpu.VMEM((rows_per_sub,), jnp.float32),
            pltpu.VMEM_SHARED((n_pad,), jnp.float32),
        ],
    )
    def deg_kernel(dst_hbm, out_hbm, idx_v, ones_v, zbuf_v, acc_s):
        c = lax.axis_index("c")
        s = lax.axis_index("s")

        def fill_ones(i, carry):
            ones_v[pl.ds(i * LANES, LANES)] = jnp.ones((LANES,), jnp.float32)
            return carry

        lax.fori_loop(0, CHUNK // LANES, fill_ones, None)

        def fill_zeros(i, carry):
            zbuf_v[pl.ds(i * LANES, LANES)] = jnp.zeros((LANES,), jnp.float32)
            return carry

        lax.fori_loop(0, rows_per_sub // LANES, fill_zeros, None)
        pltpu.sync_copy(zbuf_v, acc_s.at[pl.ds(s * rows_per_sub, rows_per_sub)])
        plsc.subcore_barrier()

        base = (c * NS + s) * edges_per_worker

        def body(ti, carry):
            pltpu.sync_copy(dst_hbm.at[pl.ds(base + ti * CHUNK, CHUNK)], idx_v)
            pltpu.sync_copy(ones_v, acc_s.at[idx_v], add=True)
            return carry

        lax.fori_loop(0, n_chunks, body, None)
        plsc.subcore_barrier()
        pltpu.sync_copy(
            acc_s.at[pl.ds(s * rows_per_sub, rows_per_sub)],
            out_hbm.at[c, pl.ds(s * rows_per_sub, rows_per_sub)],
        )

    @functools.partial(
        pl.kernel,
        out_type=jax.ShapeDtypeStruct((NC, n_pad, d), jnp.float32),
        mesh=mesh,
        scratch_types=[
            pltpu.VMEM((CHUNK,), jnp.int32),
            pltpu.VMEM((CHUNK,), jnp.int32),
            pltpu.VMEM((CHUNK, d), jnp.float32),
            pltpu.VMEM((CHUNK, d), jnp.float32),
            pltpu.VMEM_SHARED((n_pad, d), jnp.float32),
            pltpu.SemaphoreType.DMA,
        ],
    )
    def prop_kernel(hs_hbm, src_hbm, dst_hbm, out_hbm,
                    si_v, di_v, rows_v, zbuf_v, acc_s, sem):
        c = lax.axis_index("c")
        s = lax.axis_index("s")

        _fill_vmem_zeros(zbuf_v, CHUNK, d)

        def zero_acc(k, carry):
            pltpu.sync_copy(
                zbuf_v, acc_s.at[pl.ds(s * rows_per_sub + k * CHUNK, CHUNK)])
            return carry

        lax.fori_loop(0, rows_per_sub // CHUNK, zero_acc, None)
        plsc.subcore_barrier()

        base = (c * NS + s) * edges_per_worker

        def body(ti, carry):
            eb = base + ti * CHUNK
            pltpu.sync_copy(src_hbm.at[pl.ds(eb, CHUNK)], si_v)
            pltpu.sync_copy(dst_hbm.at[pl.ds(eb, CHUNK)], di_v)
            pltpu.async_copy(hs_hbm.at[si_v], rows_v, sem).wait()
            pltpu.sync_copy(rows_v, acc_s.at[di_v], add=True)
            return carry

        lax.fori_loop(0, n_chunks, body, None)
        plsc.subcore_barrier()
        pltpu.sync_copy(
            acc_s.at[pl.ds(s * rows_per_sub, rows_per_sub)],
            out_hbm.at[c, pl.ds(s * rows_per_sub, rows_per_sub)],
        )

    return deg_kernel, prop_kernel


def _mm1_body(x_ref, w_ref, d0_ref, d1_ref, hs_ref, dinv_ref):
    dinv = lax.rsqrt(d0_ref[...] + d1_ref[...] + 1.0)
    h = jnp.dot(x_ref[...], w_ref[...], preferred_element_type=jnp.float32)
    hs_ref[...] = h * dinv
    dinv_ref[...] = dinv


def _mm2_body(p_ref, hs1_ref, dinv_ref, b0_ref, w1_ref, hs2_ref):
    dinv = dinv_ref[...]
    rep = jnp.maximum(
        dinv * (p_ref[0] + p_ref[1] + hs1_ref[...]) + b0_ref[...], 0.0)
    hs2_ref[...] = jnp.dot(
        rep, w1_ref[...], preferred_element_type=jnp.float32) * dinv


def _heads_body(q_ref, hs2_ref, dinv_ref, b1_ref, t_ref,
                w00_ref, b00_ref, w10_ref, b10_ref,
                w01_ref, b01_ref, w11_ref, b11_ref,
                wpp_ref, bpp_ref, p_out, y_out):
    dinv = dinv_ref[...]
    rep = jnp.maximum(
        dinv * (q_ref[0] + q_ref[1] + hs2_ref[...]) + b1_ref[...], 0.0)
    y00 = jnp.maximum(
        jnp.dot(rep, w00_ref[...], preferred_element_type=jnp.float32)
        + b00_ref[...], 0.0)
    y10 = jnp.maximum(
        jnp.dot(rep, w10_ref[...], preferred_element_type=jnp.float32)
        + b10_ref[...], 0.0)
    y0 = jnp.dot(y00, w01_ref[...], preferred_element_type=jnp.float32) \
        + b01_ref[0, 0]
    y1 = jnp.dot(y10, w11_ref[...], preferred_element_type=jnp.float32) \
        + b11_ref[0, 0]
    y_out[...] = jnp.where(t_ref[...] > 0, y1, y0)
    p_out[...] = jax.nn.sigmoid(
        jnp.dot(rep, wpp_ref[...], preferred_element_type=jnp.float32)
        + bpp_ref[0, 0])


def kernel(x, t, z, edge_index, W_gc0, b_gc0, W_gc1, b_gc1,
           W_t00_0, b_t00_0, W_t00_1, b_t00_1,
           W_t10_0, b_t10_0, W_t10_1, b_t10_1,
           W_t01, b_t01, W_t11, b_t11, W_pp, b_pp):
    n, d = x.shape
    e = edge_index.shape[1]

    rows_unit = NS * CHUNK
    n_pad = ((n + 1 + rows_unit - 1) // rows_unit) * rows_unit
    edges_unit = NC * NS * CHUNK
    e_pad = ((e + edges_unit - 1) // edges_unit) * edges_unit

    src_p = jnp.concatenate(
        [edge_index[0], jnp.zeros((e_pad - e,), jnp.int32)])
    dst_p = jnp.concatenate(
        [edge_index[1], jnp.full((e_pad - e,), n, jnp.int32)])
    x_p = jnp.pad(x, ((0, n_pad - n), (0, 0)))
    t_p = jnp.pad(t, (0, n_pad - n)).reshape(n_pad, 1)

    deg_kernel, prop_kernel = _make_sc_kernels(n_pad, e_pad, d)

    deg = deg_kernel(dst_p)
    d0 = deg[0].reshape(n_pad, 1)
    d1 = deg[1].reshape(n_pad, 1)

    blk = 1024
    grid = (n_pad // blk,)
    row_spec = pl.BlockSpec((blk, d), lambda i: (i, 0))
    col_spec = pl.BlockSpec((blk, 1), lambda i: (i, 0))
    mat_spec = pl.BlockSpec((d, d), lambda i: (0, 0))
    bias_spec = pl.BlockSpec((1, d), lambda i: (0, 0))
    scal_spec = pl.BlockSpec((1, 1), lambda i: (0, 0))
    part_spec = pl.BlockSpec((2, blk, d), lambda i: (0, i, 0))
    vec_shape = jax.ShapeDtypeStruct((n_pad, d), jnp.float32)
    col_shape = jax.ShapeDtypeStruct((n_pad, 1), jnp.float32)

    hs1, dinv = pl.pallas_call(
        _mm1_body,
        grid=grid,
        in_specs=[row_spec, mat_spec, col_spec, col_spec],
        out_specs=[row_spec, col_spec],
        out_shape=[vec_shape, col_shape],
    )(x_p, W_gc0, d0, d1)

    p_parts = prop_kernel(hs1, src_p, dst_p)

    hs2 = pl.pallas_call(
        _mm2_body,
        grid=grid,
        in_specs=[part_spec, row_spec, col_spec, bias_spec, mat_spec],
        out_specs=row_spec,
        out_shape=vec_shape,
    )(p_parts, hs1, dinv, b_gc0.reshape(1, d), W_gc1)

    q_parts = prop_kernel(hs2, src_p, dst_p)

    one_spec = pl.BlockSpec((d, 1), lambda i: (0, 0))
    p1, y = pl.pallas_call(
        _heads_body,
        grid=grid,
        in_specs=[part_spec, row_spec, col_spec, bias_spec, col_spec,
                  mat_spec, bias_spec, mat_spec, bias_spec,
                  one_spec, scal_spec, one_spec, scal_spec,
                  one_spec, scal_spec],
        out_specs=[col_spec, col_spec],
        out_shape=[col_shape, col_shape],
    )(q_parts, hs2, dinv, b_gc1.reshape(1, d), t_p,
      W_t00_1, b_t00_1.reshape(1, d), W_t10_1, b_t10_1.reshape(1, d),
      W_t01, b_t01.reshape(1, 1), W_t11, b_t11.reshape(1, 1),
      W_pp, b_pp.reshape(1, 1))

    return (p1[:n], y[:n])
